# baseline (speedup 1.0000x reference)
import functools

import jax
import jax.numpy as jnp
from jax import lax
from jax.experimental import pallas as pl
from jax.experimental.pallas import tpu as pltpu
from jax.experimental.pallas import tpu_sc as plsc

NC = 2
NS = 16
NW = NC * NS
CHUNK = 128
WIN = 16
SC1_FRAC = 1.6 / (1.6 + 5.9)


def _make_sc_segsum(n_nodes, n_pad_rows, c_feat, ca, cb):
    rpt = n_pad_rows // NS
    last_out = n_nodes - (NS - 1) * rpt
    n_parts = 1 if cb == 0 else NC
    mesh = plsc.VectorSubcoreMesh(
        core_axis_name="c", subcore_axis_name="s", num_cores=NC, num_subcores=NS
    )

    @functools.partial(
        pl.kernel,
        mesh=mesh,
        out_type=jax.ShapeDtypeStruct((n_parts, n_nodes, c_feat), jnp.float32),
        scratch_types=[
            pltpu.VMEM_SHARED((n_pad_rows, c_feat), jnp.float32),
            pltpu.VMEM((2, WIN * CHUNK), jnp.int32),
            pltpu.VMEM((2 * WIN, CHUNK), jnp.int32),
            pltpu.VMEM((CHUNK, c_feat), jnp.float32),
            pltpu.VMEM((CHUNK, c_feat), jnp.float32),
            pltpu.SemaphoreType.DMA,
            pltpu.SemaphoreType.DMA,
            pltpu.SemaphoreType.DMA,
            pltpu.SemaphoreType.DMA,
            pltpu.SemaphoreType.DMA,
            pltpu.SemaphoreType.DMA,
        ],
    )
    def segsum(table, src2, dst2, zeros, out, acc, sidx, didx, rows0, rows1,
               gs0, gs1, ss0, ss1, is_a, is_b):
        c = lax.axis_index("c")
        s = lax.axis_index("s")
        chunk_base = jnp.where(c == 0, s * ca, NS * ca + s * cb)
        n_win = jnp.where(c == 0, ca // WIN, cb // WIN)

        rows = (rows0, rows1)
        gsem = (gs0, gs1)
        ssem = (ss0, ss1)
        isem = (is_a, is_b)

        def fetch_idx(t, half):
            wc = chunk_base + t * WIN
            pltpu.async_copy(
                src2.at[pl.ds(wc * CHUNK, WIN * CHUNK)], sidx.at[half],
                isem[half],
            )
            pltpu.async_copy(
                dst2.at[pl.ds(wc, WIN)],
                didx.at[pl.ds(half * WIN, WIN)], isem[half],
            )

        def wait_idx(half):
            pltpu.make_async_copy(
                src2.at[pl.ds(0, WIN * CHUNK)], sidx.at[half], isem[half]
            ).wait()
            pltpu.make_async_copy(
                dst2.at[pl.ds(0, WIN)], didx.at[pl.ds(half * WIN, WIN)],
                isem[half],
            ).wait()

        def gather(half, j):
            b = j % 2
            pltpu.async_copy(
                table.at[sidx.at[half].at[pl.ds(j * CHUNK, CHUNK)]],
                rows[b], gsem[b],
            )

        def wait_gather(b):
            pltpu.make_async_copy(
                table.at[sidx.at[0].at[pl.ds(0, CHUNK)]], rows[b], gsem[b]
            ).wait()

        def scatter(half, j):
            b = j % 2
            pltpu.async_copy(rows[b], acc.at[didx.at[half * WIN + j]],
                             ssem[b], add=True)

        def wait_scatter(b):
            pltpu.make_async_copy(
                rows[b], acc.at[didx.at[0]], ssem[b]
            ).wait()

        z0 = s * rpt

        @pl.when(n_win > 0)
        def _():
            fetch_idx(0, 0)

        @pl.when((c == 0) | (n_parts > 1))
        def _():
            pltpu.sync_copy(zeros.at[pl.ds(z0, rpt)], acc.at[pl.ds(z0, rpt)])

        plsc.subcore_barrier()

        def window_pair(t2, carry):
            t = 2 * t2
            for half in (0, 1):
                @pl.when(t + half + 1 < n_win)
                def _():
                    fetch_idx(t + half + 1, 1 - half)
                wait_idx(half)
                gather(half, 0)
                gather(half, 1)
                for j in range(WIN):
                    b = j % 2
                    wait_gather(b)
                    scatter(half, j)
                    if j + 2 < WIN:
                        wait_scatter(b)
                        gather(half, j + 2)
                wait_scatter(0)
                wait_scatter(1)
            return carry

        lax.fori_loop(0, n_win // 2, window_pair, 0)
        plsc.subcore_barrier()

        o0 = s * rpt

        @pl.when((c == 0) | (n_parts > 1))
        def _():
            oc = jnp.where(c < n_parts, c, 0)

            @pl.when(s < NS - 1)
            def _():
                pltpu.sync_copy(
                    acc.at[pl.ds(o0, rpt)], out.at[oc].at[pl.ds(o0, rpt)]
                )

            @pl.when(s == NS - 1)
            def _():
                base = (NS - 1) * rpt
                pltpu.sync_copy(
                    acc.at[pl.ds(base, last_out)],
                    out.at[oc].at[pl.ds(base, last_out)],
                )

    return segsum


def _tc1_body(n_parts, x_ref, p_ref, w_ref, o_ref, stats_ref, acc_ref):
    i = pl.program_id(0)
    sv = x_ref[...] + p_ref[0]
    if n_parts > 1:
        sv = sv + p_ref[1]
    o = jnp.dot(sv, w_ref[...], preferred_element_type=jnp.float32,
                precision=lax.Precision.HIGHEST)
    o_ref[...] = o
    y = jnp.maximum(o, 0.0)

    @pl.when(i == 0)
    def _():
        acc_ref[...] = jnp.zeros_like(acc_ref)

    acc_ref[0:1] += jnp.sum(y, axis=0, keepdims=True)
    acc_ref[1:2] += jnp.sum(y * y, axis=0, keepdims=True)

    @pl.when(i == pl.num_programs(0) - 1)
    def _():
        stats_ref[...] = acc_ref[...]


def _tcbn_body(n_total, o0_ref, stats_ref, g_ref, b_ref, h_ref):
    inv_n = 1.0 / n_total
    mean = stats_ref[0:1] * inv_n
    var = stats_ref[1:2] * inv_n - mean * mean
    a = g_ref[...] * lax.rsqrt(var + 1e-5)
    bb = b_ref[...] - mean * a
    y = jnp.maximum(o0_ref[...], 0.0)
    h_ref[...] = y * a + bb


def _tc3_body(n_parts, h_ref, q_ref, w_ref, o0_ref, out_ref):
    sv = h_ref[...] + q_ref[0]
    if n_parts > 1:
        sv = sv + q_ref[1]
    o = jnp.dot(sv, w_ref[...], preferred_element_type=jnp.float32,
                precision=lax.Precision.HIGHEST)
    out_ref[...] = jnp.maximum(o + o0_ref[...], 0.0)


def kernel(x, mesh, W0, W1, gamma1, beta1):
    n = x.shape[2]
    c_feat = x.shape[1]
    n_edges = mesh.shape[1]

    X = x[0, :, :, 0].T
    src = mesh[0].astype(jnp.int32)
    dst = mesh[1].astype(jnp.int32)

    tot = -(-n_edges // (NS * CHUNK))
    gran = 2 * WIN
    cb = int(round(tot * SC1_FRAC / gran)) * gran * 0
    ca = -(-max(tot - cb, 0) // gran) * gran
    n_parts = 1 if cb == 0 else NC
    e_pad = NS * (ca + cb) * CHUNK
    pad = e_pad - n_edges
    n_pad_rows = -(-(n + 1) // (NS * 8)) * (NS * 8)
    spare = n_pad_rows - n
    src_p = jnp.concatenate([src, jnp.zeros((pad,), jnp.int32)])
    dst_p = jnp.concatenate(
        [dst, n + (jnp.arange(pad, dtype=jnp.int32) % spare)]
    )
    dst2 = dst_p.reshape(e_pad // CHUNK, CHUNK)
    zeros = jnp.zeros((n_pad_rows, c_feat), jnp.float32)

    segsum = _make_sc_segsum(n, n_pad_rows, c_feat, ca, cb)

    bn = 1000
    grid = (n // bn,)
    blk = lambda i: (i, 0)
    p_spec = pl.BlockSpec((n_parts, bn, c_feat), lambda i: (0, i, 0))
    w_spec = pl.BlockSpec((c_feat, c_feat), lambda i: (0, 0))
    full_spec = pl.BlockSpec((bn, c_feat), blk)

    P = segsum(X, src_p, dst2, zeros)
    out0, stats = pl.pallas_call(
        functools.partial(_tc1_body, n_parts),
        grid=grid,
        in_specs=[full_spec, p_spec, w_spec],
        out_specs=[full_spec, pl.BlockSpec((2, c_feat), lambda i: (0, 0))],
        out_shape=[
            jax.ShapeDtypeStruct((n, c_feat), jnp.float32),
            jax.ShapeDtypeStruct((2, c_feat), jnp.float32),
        ],
        scratch_shapes=[pltpu.VMEM((2, c_feat), jnp.float32)],
    )(X, P, W0.T)

    H = pl.pallas_call(
        functools.partial(_tcbn_body, float(n)),
        grid=grid,
        in_specs=[
            full_spec,
            pl.BlockSpec((2, c_feat), lambda i: (0, 0)),
            pl.BlockSpec((1, c_feat), lambda i: (0, 0)),
            pl.BlockSpec((1, c_feat), lambda i: (0, 0)),
        ],
        out_specs=full_spec,
        out_shape=jax.ShapeDtypeStruct((n, c_feat), jnp.float32),
    )(out0, stats, gamma1.reshape(1, -1), beta1.reshape(1, -1))

    Q = segsum(H, src_p, dst2, zeros)
    F = pl.pallas_call(
        functools.partial(_tc3_body, n_parts),
        grid=grid,
        in_specs=[full_spec, p_spec, w_spec, full_spec],
        out_specs=full_spec,
        out_shape=jax.ShapeDtypeStruct((n, c_feat), jnp.float32),
    )(H, Q, W1.T, out0)

    return F.T[None, :, :, None]


# speedup vs baseline: 2.4358x; 2.4358x over previous
import functools

import jax
import jax.numpy as jnp
from jax import lax
from jax.experimental import pallas as pl
from jax.experimental.pallas import tpu as pltpu
from jax.experimental.pallas import tpu_sc as plsc

NC = 2
NS = 16
NW = NC * NS
CHUNK = 128
WIN = 16
SC1_FRAC = 1.6 / (1.6 + 5.9)


def _make_sc_segsum(n_nodes, n_pad_rows, c_feat, ca, cb):
    rpt = n_pad_rows // NS
    last_out = n_nodes - (NS - 1) * rpt
    n_parts = 1 if cb == 0 else NC
    mesh = plsc.VectorSubcoreMesh(
        core_axis_name="c", subcore_axis_name="s", num_cores=NC, num_subcores=NS
    )

    @functools.partial(
        pl.kernel,
        mesh=mesh,
        out_type=jax.ShapeDtypeStruct((n_parts, n_nodes, c_feat), jnp.float32),
        scratch_types=[
            pltpu.VMEM_SHARED((n_pad_rows, c_feat), jnp.float32),
            pltpu.VMEM((2, WIN * CHUNK), jnp.int32),
            pltpu.VMEM((2 * WIN, CHUNK), jnp.int32),
            pltpu.VMEM((CHUNK, c_feat), jnp.float32),
            pltpu.VMEM((CHUNK, c_feat), jnp.float32),
            pltpu.SemaphoreType.DMA,
            pltpu.SemaphoreType.DMA,
            pltpu.SemaphoreType.DMA,
            pltpu.SemaphoreType.DMA,
            pltpu.SemaphoreType.DMA,
            pltpu.SemaphoreType.DMA,
        ],
    )
    def segsum(table, src2, dst2, zeros, out, acc, sidx, didx, rows0, rows1,
               gs0, gs1, ss0, ss1, is_a, is_b):
        c = lax.axis_index("c")
        s = lax.axis_index("s")
        chunk_base = jnp.where(c == 0, s * ca, NS * ca + s * cb)
        n_win = jnp.where(c == 0, ca // WIN, cb // WIN)

        rows = (rows0, rows1)
        gsem = (gs0, gs1)
        ssem = (ss0, ss1)
        isem = (is_a, is_b)

        def fetch_idx(t, half):
            wc = chunk_base + t * WIN
            pltpu.async_copy(
                src2.at[pl.ds(wc * CHUNK, WIN * CHUNK)], sidx.at[half],
                isem[half],
            )
            pltpu.async_copy(
                dst2.at[pl.ds(wc, WIN)],
                didx.at[pl.ds(half * WIN, WIN)], isem[half],
            )

        def wait_idx(half):
            pltpu.make_async_copy(
                src2.at[pl.ds(0, WIN * CHUNK)], sidx.at[half], isem[half]
            ).wait()
            pltpu.make_async_copy(
                dst2.at[pl.ds(0, WIN)], didx.at[pl.ds(half * WIN, WIN)],
                isem[half],
            ).wait()

        def gather(half, j):
            b = j % 2
            pltpu.async_copy(
                table.at[sidx.at[half].at[pl.ds(j * CHUNK, CHUNK)]],
                rows[b], gsem[b],
            )

        def wait_gather(b):
            pltpu.make_async_copy(
                table.at[sidx.at[0].at[pl.ds(0, CHUNK)]], rows[b], gsem[b]
            ).wait()

        def scatter(half, j):
            b = j % 2
            pltpu.async_copy(rows[b], acc.at[didx.at[half * WIN + j]],
                             ssem[b], add=True)

        def wait_scatter(b):
            pltpu.make_async_copy(
                rows[b], acc.at[didx.at[0]], ssem[b]
            ).wait()

        z0 = s * rpt

        @pl.when(n_win > 0)
        def _():
            fetch_idx(0, 0)

        @pl.when((c == 0) | (n_parts > 1))
        def _():
            pltpu.sync_copy(zeros.at[pl.ds(z0, rpt)], acc.at[pl.ds(z0, rpt)])

        plsc.subcore_barrier()

        def window_pair(t2, carry):
            t = 2 * t2
            for half in (0, 1):
                @pl.when(t + half + 1 < n_win)
                def _():
                    fetch_idx(t + half + 1, 1 - half)
                wait_idx(half)
                gather(half, 0)
                gather(half, 1)
                for j in range(WIN):
                    b = j % 2
                    wait_gather(b)
                    scatter(half, j)
                    if j + 2 < WIN:
                        wait_scatter(b)
                        gather(half, j + 2)
                wait_scatter(0)
                wait_scatter(1)
            return carry

        lax.fori_loop(0, n_win // 2, window_pair, 0)
        plsc.subcore_barrier()

        o0 = s * rpt

        @pl.when((c == 0) | (n_parts > 1))
        def _():
            oc = jnp.where(c < n_parts, c, 0)

            @pl.when(s < NS - 1)
            def _():
                pltpu.sync_copy(
                    acc.at[pl.ds(o0, rpt)], out.at[oc].at[pl.ds(o0, rpt)]
                )

            @pl.when(s == NS - 1)
            def _():
                base = (NS - 1) * rpt
                pltpu.sync_copy(
                    acc.at[pl.ds(base, last_out)],
                    out.at[oc].at[pl.ds(base, last_out)],
                )

    return segsum


def _tc1_body(n_parts, x_ref, p_ref, w_ref, o_ref, stats_ref, acc_ref):
    i = pl.program_id(0)
    sv = x_ref[...] + p_ref[0]
    if n_parts > 1:
        sv = sv + p_ref[1]
    o = jnp.dot(sv, w_ref[...], preferred_element_type=jnp.float32,
                precision=lax.Precision.HIGHEST)
    o_ref[...] = o
    y = jnp.maximum(o, 0.0)

    @pl.when(i == 0)
    def _():
        acc_ref[...] = jnp.zeros_like(acc_ref)

    acc_ref[0:1] += jnp.sum(y, axis=0, keepdims=True)
    acc_ref[1:2] += jnp.sum(y * y, axis=0, keepdims=True)

    @pl.when(i == pl.num_programs(0) - 1)
    def _():
        stats_ref[...] = acc_ref[...]


def _tcbn_body(n_total, o0_ref, stats_ref, g_ref, b_ref, h_ref):
    inv_n = 1.0 / n_total
    mean = stats_ref[0:1] * inv_n
    var = stats_ref[1:2] * inv_n - mean * mean
    a = g_ref[...] * lax.rsqrt(var + 1e-5)
    bb = b_ref[...] - mean * a
    y = jnp.maximum(o0_ref[...], 0.0)
    h_ref[...] = y * a + bb


def _tc3_body(n_parts, h_ref, q_ref, w_ref, o0_ref, out_ref):
    sv = h_ref[...] + q_ref[0]
    if n_parts > 1:
        sv = sv + q_ref[1]
    o = jnp.dot(sv, w_ref[...], preferred_element_type=jnp.float32,
                precision=lax.Precision.HIGHEST)
    out_ref[...] = jnp.maximum(o + o0_ref[...], 0.0)


def kernel(x, mesh, W0, W1, gamma1, beta1):
    n = x.shape[2]
    c_feat = x.shape[1]
    n_edges = mesh.shape[1]

    X = x[0, :, :, 0].T
    src = mesh[0].astype(jnp.int32)
    dst = mesh[1].astype(jnp.int32)

    tot = -(-n_edges // (NS * CHUNK))
    gran = 2 * WIN
    cb = int(round(tot * SC1_FRAC / gran)) * gran * 0
    ca = -(-max(tot - cb, 0) // gran) * gran
    n_parts = 1 if cb == 0 else NC
    e_pad = NS * (ca + cb) * CHUNK
    pad = e_pad - n_edges
    n_pad_rows = -(-(n + 1) // (NS * 8)) * (NS * 8)
    spare = n_pad_rows - n
    pad_iota = jnp.arange(pad, dtype=jnp.int32)
    src_p = jnp.concatenate([src, (pad_iota * 79) % n])
    dst_p = jnp.concatenate([dst, n + (pad_iota % spare)])
    dst2 = dst_p.reshape(e_pad // CHUNK, CHUNK)
    zeros = jnp.zeros((n_pad_rows, c_feat), jnp.float32)

    segsum = _make_sc_segsum(n, n_pad_rows, c_feat, ca, cb)

    bn = 1000
    grid = (n // bn,)
    blk = lambda i: (i, 0)
    p_spec = pl.BlockSpec((n_parts, bn, c_feat), lambda i: (0, i, 0))
    w_spec = pl.BlockSpec((c_feat, c_feat), lambda i: (0, 0))
    full_spec = pl.BlockSpec((bn, c_feat), blk)

    P = segsum(X, src_p, dst2, zeros)
    out0, stats = pl.pallas_call(
        functools.partial(_tc1_body, n_parts),
        grid=grid,
        in_specs=[full_spec, p_spec, w_spec],
        out_specs=[full_spec, pl.BlockSpec((2, c_feat), lambda i: (0, 0))],
        out_shape=[
            jax.ShapeDtypeStruct((n, c_feat), jnp.float32),
            jax.ShapeDtypeStruct((2, c_feat), jnp.float32),
        ],
        scratch_shapes=[pltpu.VMEM((2, c_feat), jnp.float32)],
    )(X, P, W0.T)

    H = pl.pallas_call(
        functools.partial(_tcbn_body, float(n)),
        grid=grid,
        in_specs=[
            full_spec,
            pl.BlockSpec((2, c_feat), lambda i: (0, 0)),
            pl.BlockSpec((1, c_feat), lambda i: (0, 0)),
            pl.BlockSpec((1, c_feat), lambda i: (0, 0)),
        ],
        out_specs=full_spec,
        out_shape=jax.ShapeDtypeStruct((n, c_feat), jnp.float32),
    )(out0, stats, gamma1.reshape(1, -1), beta1.reshape(1, -1))

    Q = segsum(H, src_p, dst2, zeros)
    F = pl.pallas_call(
        functools.partial(_tc3_body, n_parts),
        grid=grid,
        in_specs=[full_spec, p_spec, w_spec, full_spec],
        out_specs=full_spec,
        out_shape=jax.ShapeDtypeStruct((n, c_feat), jnp.float32),
    )(H, Q, W1.T, out0)

    return F.T[None, :, :, None]


# speedup vs baseline: 3.4971x; 1.4357x over previous
import functools

import jax
import jax.numpy as jnp
from jax import lax
from jax.experimental import pallas as pl
from jax.experimental.pallas import tpu as pltpu
from jax.experimental.pallas import tpu_sc as plsc

NC = 2
NS = 16
NW = NC * NS
CHUNK = 128
WIN = 16
SC1_FRAC = 0.4


def _make_sc_segsum(n_nodes, n_pad_rows, c_feat, ca, cb):
    rpt = n_pad_rows // NS
    last_out = n_nodes - (NS - 1) * rpt
    n_parts = 1 if cb == 0 else NC
    mesh = plsc.VectorSubcoreMesh(
        core_axis_name="c", subcore_axis_name="s", num_cores=NC, num_subcores=NS
    )

    @functools.partial(
        pl.kernel,
        mesh=mesh,
        out_type=jax.ShapeDtypeStruct((n_parts, n_nodes, c_feat), jnp.float32),
        scratch_types=[
            pltpu.VMEM_SHARED((n_pad_rows, c_feat), jnp.float32),
            pltpu.VMEM((2, WIN * CHUNK), jnp.int32),
            pltpu.VMEM((2 * WIN, CHUNK), jnp.int32),
            pltpu.VMEM((CHUNK, c_feat), jnp.float32),
            pltpu.VMEM((CHUNK, c_feat), jnp.float32),
            pltpu.SemaphoreType.DMA,
            pltpu.SemaphoreType.DMA,
            pltpu.SemaphoreType.DMA,
            pltpu.SemaphoreType.DMA,
            pltpu.SemaphoreType.DMA,
            pltpu.SemaphoreType.DMA,
        ],
    )
    def segsum(table, src2, dst2, zeros, out, acc, sidx, didx, rows0, rows1,
               gs0, gs1, ss0, ss1, is_a, is_b):
        c = lax.axis_index("c")
        s = lax.axis_index("s")
        chunk_base = jnp.where(c == 0, s * ca, NS * ca + s * cb)
        n_win = jnp.where(c == 0, ca // WIN, cb // WIN)

        rows = (rows0, rows1)
        gsem = (gs0, gs1)
        ssem = (ss0, ss1)
        isem = (is_a, is_b)

        def fetch_idx(t, half):
            wc = chunk_base + t * WIN
            pltpu.async_copy(
                src2.at[pl.ds(wc * CHUNK, WIN * CHUNK)], sidx.at[half],
                isem[half],
            )
            pltpu.async_copy(
                dst2.at[pl.ds(wc, WIN)],
                didx.at[pl.ds(half * WIN, WIN)], isem[half],
            )

        def wait_idx(half):
            pltpu.make_async_copy(
                src2.at[pl.ds(0, WIN * CHUNK)], sidx.at[half], isem[half]
            ).wait()
            pltpu.make_async_copy(
                dst2.at[pl.ds(0, WIN)], didx.at[pl.ds(half * WIN, WIN)],
                isem[half],
            ).wait()

        def gather(half, j):
            b = j % 2
            pltpu.async_copy(
                table.at[sidx.at[half].at[pl.ds(j * CHUNK, CHUNK)]],
                rows[b], gsem[b],
            )

        def wait_gather(b):
            pltpu.make_async_copy(
                table.at[sidx.at[0].at[pl.ds(0, CHUNK)]], rows[b], gsem[b]
            ).wait()

        def scatter(half, j):
            b = j % 2
            pltpu.async_copy(rows[b], acc.at[didx.at[half * WIN + j]],
                             ssem[b], add=True)

        def wait_scatter(b):
            pltpu.make_async_copy(
                rows[b], acc.at[didx.at[0]], ssem[b]
            ).wait()

        z0 = s * rpt

        @pl.when(n_win > 0)
        def _():
            fetch_idx(0, 0)

        @pl.when((c == 0) | (n_parts > 1))
        def _():
            pltpu.sync_copy(zeros.at[pl.ds(z0, rpt)], acc.at[pl.ds(z0, rpt)])

        plsc.subcore_barrier()

        def window_pair(t2, carry):
            t = 2 * t2
            for half in (0, 1):
                @pl.when(t + half + 1 < n_win)
                def _():
                    fetch_idx(t + half + 1, 1 - half)
                wait_idx(half)
                gather(half, 0)
                gather(half, 1)
                for j in range(WIN):
                    b = j % 2
                    wait_gather(b)
                    scatter(half, j)
                    if j + 2 < WIN:
                        wait_scatter(b)
                        gather(half, j + 2)
                wait_scatter(0)
                wait_scatter(1)
            return carry

        lax.fori_loop(0, n_win // 2, window_pair, 0)
        plsc.subcore_barrier()

        o0 = s * rpt

        @pl.when((c == 0) | (n_parts > 1))
        def _():
            oc = jnp.where(c < n_parts, c, 0)

            @pl.when(s < NS - 1)
            def _():
                pltpu.sync_copy(
                    acc.at[pl.ds(o0, rpt)], out.at[oc].at[pl.ds(o0, rpt)]
                )

            @pl.when(s == NS - 1)
            def _():
                base = (NS - 1) * rpt
                pltpu.sync_copy(
                    acc.at[pl.ds(base, last_out)],
                    out.at[oc].at[pl.ds(base, last_out)],
                )

    return segsum


def _tc1_body(n_parts, x_ref, p_ref, w_ref, o_ref, stats_ref, acc_ref):
    i = pl.program_id(0)
    sv = x_ref[...] + p_ref[0]
    if n_parts > 1:
        sv = sv + p_ref[1]
    o = jnp.dot(sv, w_ref[...], preferred_element_type=jnp.float32,
                precision=lax.Precision.HIGHEST)
    o_ref[...] = o
    y = jnp.maximum(o, 0.0)

    @pl.when(i == 0)
    def _():
        acc_ref[...] = jnp.zeros_like(acc_ref)

    acc_ref[0:1] += jnp.sum(y, axis=0, keepdims=True)
    acc_ref[1:2] += jnp.sum(y * y, axis=0, keepdims=True)

    @pl.when(i == pl.num_programs(0) - 1)
    def _():
        stats_ref[...] = acc_ref[...]


def _tcbn_body(n_total, o0_ref, stats_ref, g_ref, b_ref, h_ref):
    inv_n = 1.0 / n_total
    mean = stats_ref[0:1] * inv_n
    var = stats_ref[1:2] * inv_n - mean * mean
    a = g_ref[...] * lax.rsqrt(var + 1e-5)
    bb = b_ref[...] - mean * a
    y = jnp.maximum(o0_ref[...], 0.0)
    h_ref[...] = y * a + bb


def _tc3_body(n_parts, h_ref, q_ref, w_ref, o0_ref, out_ref):
    sv = h_ref[...] + q_ref[0]
    if n_parts > 1:
        sv = sv + q_ref[1]
    o = jnp.dot(sv, w_ref[...], preferred_element_type=jnp.float32,
                precision=lax.Precision.HIGHEST)
    out_ref[...] = jnp.maximum(o + o0_ref[...], 0.0)


def kernel(x, mesh, W0, W1, gamma1, beta1):
    n = x.shape[2]
    c_feat = x.shape[1]
    n_edges = mesh.shape[1]

    X = x[0, :, :, 0].T
    src = mesh[0].astype(jnp.int32)
    dst = mesh[1].astype(jnp.int32)

    tot = -(-n_edges // (NS * CHUNK))
    gran = 2 * WIN
    cb = int(round(tot * SC1_FRAC / gran)) * gran
    ca = -(-max(tot - cb, 0) // gran) * gran
    n_parts = 1 if cb == 0 else NC
    e_pad = NS * (ca + cb) * CHUNK
    pad = e_pad - n_edges
    n_pad_rows = -(-(n + 1) // (NS * 8)) * (NS * 8)
    spare = n_pad_rows - n
    pad_iota = jnp.arange(pad, dtype=jnp.int32)
    src_p = jnp.concatenate([src, (pad_iota * 79) % n])
    dst_p = jnp.concatenate([dst, n + (pad_iota % spare)])
    dst2 = dst_p.reshape(e_pad // CHUNK, CHUNK)
    zeros = jnp.zeros((n_pad_rows, c_feat), jnp.float32)

    segsum = _make_sc_segsum(n, n_pad_rows, c_feat, ca, cb)

    bn = 1000
    grid = (n // bn,)
    blk = lambda i: (i, 0)
    p_spec = pl.BlockSpec((n_parts, bn, c_feat), lambda i: (0, i, 0))
    w_spec = pl.BlockSpec((c_feat, c_feat), lambda i: (0, 0))
    full_spec = pl.BlockSpec((bn, c_feat), blk)

    P = segsum(X, src_p, dst2, zeros)
    out0, stats = pl.pallas_call(
        functools.partial(_tc1_body, n_parts),
        grid=grid,
        in_specs=[full_spec, p_spec, w_spec],
        out_specs=[full_spec, pl.BlockSpec((2, c_feat), lambda i: (0, 0))],
        out_shape=[
            jax.ShapeDtypeStruct((n, c_feat), jnp.float32),
            jax.ShapeDtypeStruct((2, c_feat), jnp.float32),
        ],
        scratch_shapes=[pltpu.VMEM((2, c_feat), jnp.float32)],
    )(X, P, W0.T)

    H = pl.pallas_call(
        functools.partial(_tcbn_body, float(n)),
        grid=grid,
        in_specs=[
            full_spec,
            pl.BlockSpec((2, c_feat), lambda i: (0, 0)),
            pl.BlockSpec((1, c_feat), lambda i: (0, 0)),
            pl.BlockSpec((1, c_feat), lambda i: (0, 0)),
        ],
        out_specs=full_spec,
        out_shape=jax.ShapeDtypeStruct((n, c_feat), jnp.float32),
    )(out0, stats, gamma1.reshape(1, -1), beta1.reshape(1, -1))

    Q = segsum(H, src_p, dst2, zeros)
    F = pl.pallas_call(
        functools.partial(_tc3_body, n_parts),
        grid=grid,
        in_specs=[full_spec, p_spec, w_spec, full_spec],
        out_specs=full_spec,
        out_shape=jax.ShapeDtypeStruct((n, c_feat), jnp.float32),
    )(H, Q, W1.T, out0)

    return F.T[None, :, :, None]


# speedup vs baseline: 3.7738x; 1.0791x over previous
import functools

import jax
import jax.numpy as jnp
from jax import lax
from jax.experimental import pallas as pl
from jax.experimental.pallas import tpu as pltpu
from jax.experimental.pallas import tpu_sc as plsc

NC = 2
NS = 16
NW = NC * NS
CHUNK = 128
WIN = 8
SC1_FRAC = 0.5


def _make_sc_segsum(n_nodes, n_pad_rows, c_feat, ca, cb):
    rpt = n_pad_rows // NS
    last_out = n_nodes - (NS - 1) * rpt
    n_parts = 1 if cb == 0 else NC
    mesh = plsc.VectorSubcoreMesh(
        core_axis_name="c", subcore_axis_name="s", num_cores=NC, num_subcores=NS
    )

    @functools.partial(
        pl.kernel,
        mesh=mesh,
        out_type=jax.ShapeDtypeStruct((n_parts, n_nodes, c_feat), jnp.float32),
        scratch_types=[
            pltpu.VMEM_SHARED((n_pad_rows, c_feat), jnp.float32),
            pltpu.VMEM((2, WIN * CHUNK), jnp.int32),
            pltpu.VMEM((2 * WIN, CHUNK), jnp.int32),
            pltpu.VMEM((CHUNK, c_feat), jnp.float32),
            pltpu.VMEM((CHUNK, c_feat), jnp.float32),
            pltpu.SemaphoreType.DMA,
            pltpu.SemaphoreType.DMA,
            pltpu.SemaphoreType.DMA,
            pltpu.SemaphoreType.DMA,
            pltpu.SemaphoreType.DMA,
            pltpu.SemaphoreType.DMA,
        ],
    )
    def segsum(table, src2, dst2, zeros, out, acc, sidx, didx, rows0, rows1,
               gs0, gs1, ss0, ss1, is_a, is_b):
        c = lax.axis_index("c")
        s = lax.axis_index("s")
        chunk_base = jnp.where(c == 0, s * ca, NS * ca + s * cb)
        n_win = jnp.where(c == 0, ca // WIN, cb // WIN)

        rows = (rows0, rows1)
        gsem = (gs0, gs1)
        ssem = (ss0, ss1)
        isem = (is_a, is_b)

        def fetch_idx(t, half):
            wc = chunk_base + t * WIN
            pltpu.async_copy(
                src2.at[pl.ds(wc * CHUNK, WIN * CHUNK)], sidx.at[half],
                isem[half],
            )
            pltpu.async_copy(
                dst2.at[pl.ds(wc, WIN)],
                didx.at[pl.ds(half * WIN, WIN)], isem[half],
            )

        def wait_idx(half):
            pltpu.make_async_copy(
                src2.at[pl.ds(0, WIN * CHUNK)], sidx.at[half], isem[half]
            ).wait()
            pltpu.make_async_copy(
                dst2.at[pl.ds(0, WIN)], didx.at[pl.ds(half * WIN, WIN)],
                isem[half],
            ).wait()

        def gather(half, j):
            b = j % 2
            pltpu.async_copy(
                table.at[sidx.at[half].at[pl.ds(j * CHUNK, CHUNK)]],
                rows[b], gsem[b],
            )

        def wait_gather(b):
            pltpu.make_async_copy(
                table.at[sidx.at[0].at[pl.ds(0, CHUNK)]], rows[b], gsem[b]
            ).wait()

        def scatter(half, j):
            b = j % 2
            pltpu.async_copy(rows[b], acc.at[didx.at[half * WIN + j]],
                             ssem[b], add=True)

        def wait_scatter(b):
            pltpu.make_async_copy(
                rows[b], acc.at[didx.at[0]], ssem[b]
            ).wait()

        z0 = s * rpt

        @pl.when(n_win > 0)
        def _():
            fetch_idx(0, 0)

        @pl.when((c == 0) | (n_parts > 1))
        def _():
            pltpu.sync_copy(zeros.at[pl.ds(z0, rpt)], acc.at[pl.ds(z0, rpt)])

        plsc.subcore_barrier()

        def window_pair(t2, carry):
            t = 2 * t2
            for half in (0, 1):
                @pl.when(t + half + 1 < n_win)
                def _():
                    fetch_idx(t + half + 1, 1 - half)
                wait_idx(half)
                gather(half, 0)
                gather(half, 1)
                for j in range(WIN):
                    b = j % 2
                    wait_gather(b)
                    scatter(half, j)
                    if j + 2 < WIN:
                        wait_scatter(b)
                        gather(half, j + 2)
                wait_scatter(0)
                wait_scatter(1)
            return carry

        lax.fori_loop(0, n_win // 2, window_pair, 0)
        plsc.subcore_barrier()

        o0 = s * rpt

        @pl.when((c == 0) | (n_parts > 1))
        def _():
            oc = jnp.where(c < n_parts, c, 0)

            @pl.when(s < NS - 1)
            def _():
                pltpu.sync_copy(
                    acc.at[pl.ds(o0, rpt)], out.at[oc].at[pl.ds(o0, rpt)]
                )

            @pl.when(s == NS - 1)
            def _():
                base = (NS - 1) * rpt
                pltpu.sync_copy(
                    acc.at[pl.ds(base, last_out)],
                    out.at[oc].at[pl.ds(base, last_out)],
                )

    return segsum


def _tc1_body(n_parts, x_ref, p_ref, w_ref, o_ref, stats_ref, acc_ref):
    i = pl.program_id(0)
    sv = x_ref[...] + p_ref[0]
    if n_parts > 1:
        sv = sv + p_ref[1]
    o = jnp.dot(sv, w_ref[...], preferred_element_type=jnp.float32,
                precision=lax.Precision.HIGHEST)
    o_ref[...] = o
    y = jnp.maximum(o, 0.0)

    @pl.when(i == 0)
    def _():
        acc_ref[...] = jnp.zeros_like(acc_ref)

    acc_ref[0:1] += jnp.sum(y, axis=0, keepdims=True)
    acc_ref[1:2] += jnp.sum(y * y, axis=0, keepdims=True)

    @pl.when(i == pl.num_programs(0) - 1)
    def _():
        stats_ref[...] = acc_ref[...]


def _tcbn_body(n_total, o0_ref, stats_ref, g_ref, b_ref, h_ref):
    inv_n = 1.0 / n_total
    mean = stats_ref[0:1] * inv_n
    var = stats_ref[1:2] * inv_n - mean * mean
    a = g_ref[...] * lax.rsqrt(var + 1e-5)
    bb = b_ref[...] - mean * a
    y = jnp.maximum(o0_ref[...], 0.0)
    h_ref[...] = y * a + bb


def _tc3_body(n_parts, h_ref, q_ref, w_ref, o0_ref, out_ref):
    sv = h_ref[...] + q_ref[0]
    if n_parts > 1:
        sv = sv + q_ref[1]
    o = jnp.dot(sv, w_ref[...], preferred_element_type=jnp.float32,
                precision=lax.Precision.HIGHEST)
    out_ref[...] = jnp.maximum(o + o0_ref[...], 0.0)


def kernel(x, mesh, W0, W1, gamma1, beta1):
    n = x.shape[2]
    c_feat = x.shape[1]
    n_edges = mesh.shape[1]

    X = x[0, :, :, 0].T
    src = mesh[0].astype(jnp.int32)
    dst = mesh[1].astype(jnp.int32)

    tot = -(-n_edges // (NS * CHUNK))
    gran = 2 * WIN
    cb = int(round(tot * SC1_FRAC / gran)) * gran
    ca = -(-max(tot - cb, 0) // gran) * gran
    n_parts = 1 if cb == 0 else NC
    e_pad = NS * (ca + cb) * CHUNK
    pad = e_pad - n_edges
    n_pad_rows = -(-(n + 1) // (NS * 8)) * (NS * 8)
    spare = n_pad_rows - n
    pad_iota = jnp.arange(pad, dtype=jnp.int32)
    src_p = jnp.concatenate([src, (pad_iota * 79) % n])
    dst_p = jnp.concatenate([dst, n + (pad_iota % spare)])
    dst2 = dst_p.reshape(e_pad // CHUNK, CHUNK)
    zeros = jnp.zeros((n_pad_rows, c_feat), jnp.float32)

    segsum = _make_sc_segsum(n, n_pad_rows, c_feat, ca, cb)

    bn = 1000
    grid = (n // bn,)
    blk = lambda i: (i, 0)
    p_spec = pl.BlockSpec((n_parts, bn, c_feat), lambda i: (0, i, 0))
    w_spec = pl.BlockSpec((c_feat, c_feat), lambda i: (0, 0))
    full_spec = pl.BlockSpec((bn, c_feat), blk)

    P = segsum(X, src_p, dst2, zeros)
    out0, stats = pl.pallas_call(
        functools.partial(_tc1_body, n_parts),
        grid=grid,
        in_specs=[full_spec, p_spec, w_spec],
        out_specs=[full_spec, pl.BlockSpec((2, c_feat), lambda i: (0, 0))],
        out_shape=[
            jax.ShapeDtypeStruct((n, c_feat), jnp.float32),
            jax.ShapeDtypeStruct((2, c_feat), jnp.float32),
        ],
        scratch_shapes=[pltpu.VMEM((2, c_feat), jnp.float32)],
    )(X, P, W0.T)

    H = pl.pallas_call(
        functools.partial(_tcbn_body, float(n)),
        grid=grid,
        in_specs=[
            full_spec,
            pl.BlockSpec((2, c_feat), lambda i: (0, 0)),
            pl.BlockSpec((1, c_feat), lambda i: (0, 0)),
            pl.BlockSpec((1, c_feat), lambda i: (0, 0)),
        ],
        out_specs=full_spec,
        out_shape=jax.ShapeDtypeStruct((n, c_feat), jnp.float32),
    )(out0, stats, gamma1.reshape(1, -1), beta1.reshape(1, -1))

    Q = segsum(H, src_p, dst2, zeros)
    F = pl.pallas_call(
        functools.partial(_tc3_body, n_parts),
        grid=grid,
        in_specs=[full_spec, p_spec, w_spec, full_spec],
        out_specs=full_spec,
        out_shape=jax.ShapeDtypeStruct((n, c_feat), jnp.float32),
    )(H, Q, W1.T, out0)

    return F.T[None, :, :, None]


# speedup vs baseline: 3.8896x; 1.0307x over previous
import functools

import jax
import jax.numpy as jnp
from jax import lax
from jax.experimental import pallas as pl
from jax.experimental.pallas import tpu as pltpu
from jax.experimental.pallas import tpu_sc as plsc

NC = 2
NS = 16
NW = NC * NS
CHUNK = 128
WIN = 8
SC1_FRAC = 0.5


def _make_sc_segsum(n_nodes, n_pad_rows, c_feat, ca, cb):
    rpt = n_pad_rows // NS
    last_out = n_nodes - (NS - 1) * rpt
    n_parts = 1 if cb == 0 else NC
    mesh = plsc.VectorSubcoreMesh(
        core_axis_name="c", subcore_axis_name="s", num_cores=NC, num_subcores=NS
    )

    @functools.partial(
        pl.kernel,
        mesh=mesh,
        out_type=jax.ShapeDtypeStruct((n_parts, n_nodes, c_feat), jnp.float32),
        scratch_types=[
            pltpu.VMEM_SHARED((n_pad_rows, c_feat), jnp.float32),
            pltpu.VMEM((2, WIN * CHUNK), jnp.int32),
            pltpu.VMEM((2 * WIN, CHUNK), jnp.int32),
            pltpu.VMEM((CHUNK, c_feat), jnp.float32),
            pltpu.VMEM((CHUNK, c_feat), jnp.float32),
            pltpu.SemaphoreType.DMA,
            pltpu.SemaphoreType.DMA,
            pltpu.SemaphoreType.DMA,
            pltpu.SemaphoreType.DMA,
            pltpu.SemaphoreType.DMA,
            pltpu.SemaphoreType.DMA,
        ],
    )
    def segsum(table, src2, dst2, zeros, out, acc, sidx, didx, rows0, rows1,
               gs0, gs1, ss0, ss1, is_a, is_b):
        c = lax.axis_index("c")
        s = lax.axis_index("s")
        chunk_base = jnp.where(c == 0, s * ca, NS * ca + s * cb)
        n_win = jnp.where(c == 0, ca // WIN, cb // WIN)

        rows = (rows0, rows1)
        gsem = (gs0, gs1)
        ssem = (ss0, ss1)
        isem = (is_a, is_b)

        def fetch_idx(t, half):
            wc = chunk_base + t * WIN
            pltpu.async_copy(
                src2.at[pl.ds(wc * CHUNK, WIN * CHUNK)], sidx.at[half],
                isem[half],
            )
            pltpu.async_copy(
                dst2.at[pl.ds(wc, WIN)],
                didx.at[pl.ds(half * WIN, WIN)], isem[half],
            )

        def wait_idx(half):
            pltpu.make_async_copy(
                src2.at[pl.ds(0, WIN * CHUNK)], sidx.at[half], isem[half]
            ).wait()
            pltpu.make_async_copy(
                dst2.at[pl.ds(0, WIN)], didx.at[pl.ds(half * WIN, WIN)],
                isem[half],
            ).wait()

        def gather(half, j):
            b = j % 2
            pltpu.async_copy(
                table.at[sidx.at[half].at[pl.ds(j * CHUNK, CHUNK)]],
                rows[b], gsem[b],
            )

        def wait_gather(b):
            pltpu.make_async_copy(
                table.at[sidx.at[0].at[pl.ds(0, CHUNK)]], rows[b], gsem[b]
            ).wait()

        def scatter(half, j):
            b = j % 2
            pltpu.async_copy(rows[b], acc.at[didx.at[half * WIN + j]],
                             ssem[b], add=True)

        def wait_scatter(b):
            pltpu.make_async_copy(
                rows[b], acc.at[didx.at[0]], ssem[b]
            ).wait()

        z0 = s * rpt

        @pl.when(n_win > 0)
        def _():
            fetch_idx(0, 0)

        @pl.when((c == 0) | (n_parts > 1))
        def _():
            pltpu.sync_copy(zeros.at[pl.ds(z0, rpt)], acc.at[pl.ds(z0, rpt)])

        plsc.subcore_barrier()

        def window_pair(t2, carry):
            t = 2 * t2
            for half in (0, 1):
                @pl.when(t + half + 1 < n_win)
                def _():
                    fetch_idx(t + half + 1, 1 - half)
                wait_idx(half)
                gather(half, 0)
                gather(half, 1)
                for j in range(WIN):
                    b = j % 2
                    wait_gather(b)
                    scatter(half, j)
                    if j + 2 < WIN:
                        wait_scatter(b)
                        gather(half, j + 2)
                wait_scatter(0)
                wait_scatter(1)
            return carry

        lax.fori_loop(0, n_win // 2, window_pair, 0)
        plsc.subcore_barrier()

        o0 = s * rpt

        @pl.when((c == 0) | (n_parts > 1))
        def _():
            oc = jnp.where(c < n_parts, c, 0)

            @pl.when(s < NS - 1)
            def _():
                pltpu.sync_copy(
                    acc.at[pl.ds(o0, rpt)], out.at[oc].at[pl.ds(o0, rpt)]
                )

            @pl.when(s == NS - 1)
            def _():
                base = (NS - 1) * rpt
                pltpu.sync_copy(
                    acc.at[pl.ds(base, last_out)],
                    out.at[oc].at[pl.ds(base, last_out)],
                )

    return segsum


def _tc1_body(n_parts, x_ref, p_ref, w_ref, o_ref, stats_ref, acc_ref):
    i = pl.program_id(0)
    sv = x_ref[...] + p_ref[0]
    if n_parts > 1:
        sv = sv + p_ref[1]
    o = jnp.dot(sv, w_ref[...], preferred_element_type=jnp.float32,
                precision=lax.Precision.HIGHEST)
    o_ref[...] = o
    y = jnp.maximum(o, 0.0)

    @pl.when(i == 0)
    def _():
        acc_ref[...] = jnp.zeros_like(acc_ref)

    acc_ref[0:1] += jnp.sum(y, axis=0, keepdims=True)
    acc_ref[1:2] += jnp.sum(y * y, axis=0, keepdims=True)

    @pl.when(i == pl.num_programs(0) - 1)
    def _():
        stats_ref[...] = acc_ref[...]


def _tcbn_body(n_total, o0_ref, stats_ref, g_ref, b_ref, h_ref):
    inv_n = 1.0 / n_total
    mean = stats_ref[0:1] * inv_n
    var = stats_ref[1:2] * inv_n - mean * mean
    a = g_ref[...] * lax.rsqrt(var + 1e-5)
    bb = b_ref[...] - mean * a
    y = jnp.maximum(o0_ref[...], 0.0)
    h_ref[...] = y * a + bb


def _tc3_body(n_parts, h_ref, q_ref, w_ref, o0_ref, out_ref):
    sv = h_ref[...] + q_ref[0]
    if n_parts > 1:
        sv = sv + q_ref[1]
    o = jnp.dot(sv, w_ref[...], preferred_element_type=jnp.float32,
                precision=lax.Precision.HIGHEST)
    out_ref[...] = jnp.maximum(o + o0_ref[...], 0.0)


def kernel(x, mesh, W0, W1, gamma1, beta1):
    n = x.shape[2]
    c_feat = x.shape[1]
    n_edges = mesh.shape[1]

    X = x[0, :, :, 0].T
    src = mesh[0].astype(jnp.int32)
    dst = mesh[1].astype(jnp.int32)

    tot = -(-n_edges // (NS * CHUNK))
    gran = 2 * WIN
    cb = int(round(tot * SC1_FRAC / gran)) * gran
    ca = -(-max(tot - cb, 0) // gran) * gran
    n_parts = 1 if cb == 0 else NC
    e_pad = NS * (ca + cb) * CHUNK
    pad = e_pad - n_edges
    n_pad_rows = -(-(n + 1) // (NS * 8)) * (NS * 8)
    spare = n_pad_rows - n
    pad_iota = jnp.arange(pad, dtype=jnp.int32)
    src_p = jnp.concatenate([src, (pad_iota * 79) % n])
    dst_p = jnp.concatenate([dst, n + (pad_iota % spare)])
    dst2 = dst_p.reshape(e_pad // CHUNK, CHUNK)
    zeros = jnp.zeros((n_pad_rows, c_feat), jnp.float32)

    segsum = _make_sc_segsum(n, n_pad_rows, c_feat, ca, cb)

    bn = 2000 if n % 2000 == 0 else 1000
    grid = (n // bn,)
    blk = lambda i: (i, 0)
    p_spec = pl.BlockSpec((n_parts, bn, c_feat), lambda i: (0, i, 0))
    w_spec = pl.BlockSpec((c_feat, c_feat), lambda i: (0, 0))
    full_spec = pl.BlockSpec((bn, c_feat), blk)

    P = segsum(X, src_p, dst2, zeros)
    out0, stats = pl.pallas_call(
        functools.partial(_tc1_body, n_parts),
        grid=grid,
        in_specs=[full_spec, p_spec, w_spec],
        out_specs=[full_spec, pl.BlockSpec((2, c_feat), lambda i: (0, 0))],
        out_shape=[
            jax.ShapeDtypeStruct((n, c_feat), jnp.float32),
            jax.ShapeDtypeStruct((2, c_feat), jnp.float32),
        ],
        scratch_shapes=[pltpu.VMEM((2, c_feat), jnp.float32)],
    )(X, P, W0.T)

    H = pl.pallas_call(
        functools.partial(_tcbn_body, float(n)),
        grid=grid,
        in_specs=[
            full_spec,
            pl.BlockSpec((2, c_feat), lambda i: (0, 0)),
            pl.BlockSpec((1, c_feat), lambda i: (0, 0)),
            pl.BlockSpec((1, c_feat), lambda i: (0, 0)),
        ],
        out_specs=full_spec,
        out_shape=jax.ShapeDtypeStruct((n, c_feat), jnp.float32),
    )(out0, stats, gamma1.reshape(1, -1), beta1.reshape(1, -1))

    Q = segsum(H, src_p, dst2, zeros)
    F = pl.pallas_call(
        functools.partial(_tc3_body, n_parts),
        grid=grid,
        in_specs=[full_spec, p_spec, w_spec, full_spec],
        out_specs=full_spec,
        out_shape=jax.ShapeDtypeStruct((n, c_feat), jnp.float32),
    )(H, Q, W1.T, out0)

    return F.T[None, :, :, None]
